# baseline (speedup 1.0000x reference)
import functools

import jax
import jax.numpy as jnp
from jax import lax
from jax.experimental import pallas as pl
from jax.experimental.pallas import tpu as pltpu
from jax.experimental.pallas import tpu_sc as plsc

N = 10000
E = 320000
D = 128
DH = D // 2

NC = 2
NS = 16
CHUNK = 128
NCH = E // CHUNK
BASE_CH = NCH // NS
EXTRA = NCH % NS
TMAX = (BASE_CH + 1 + 2 + 3) // 4
NP = 10240
RPT = NP // NS


def _agg_body(x2_hbm, src_hbm, nbr_hbm, ones_hbm,
              sum_a_hbm, sum_b_hbm, cnt_a_hbm, cnt_b_hbm,
              s0, s1, s2, s3, n0, n1, n2, n3, i0, i1, i2, i3,
              r0, r1, stage, ones_v, st16,
              semi0, semi1, semi2, semi3, semg, sems0, sems1,
              acc, cnt):
    cid = lax.axis_index("c")
    sid = lax.axis_index("s")
    srcb = [s0, s1, s2, s3]
    nbrb = [n0, n1, n2, n3]
    idxb = [i0, i1, i2, i3]
    rowsb = [r0, r1]
    semi = [semi0, semi1, semi2, semi3]
    sems = [sems0, sems1]
    nch = BASE_CH + jnp.where(sid < EXTRA, 1, 0)

    zv = jnp.zeros((16,), jnp.float32)
    zvb = jnp.zeros((32,), jnp.bfloat16)

    def zrow(i, carry):
        for j in range(DH // 32):
            stage[i, pl.ds(j * 32, 32)] = zvb
        st16[i] = zv
        return carry
    lax.fori_loop(0, RPT, zrow, 0)
    pltpu.sync_copy(stage, acc.at[pl.ds(sid * RPT, RPT)])
    pltpu.sync_copy(st16, cnt.at[pl.ds(sid * RPT, RPT)])
    pltpu.sync_copy(ones_hbm, ones_v)
    plsc.subcore_barrier()

    def fire_loads(s, u):
        base = (s * NS + sid) * CHUNK
        sl = pl.ds(base, CHUNK)
        pltpu.async_copy(src_hbm.at[sl], srcb[u], semi[u])
        pltpu.async_copy(nbr_hbm.at[sl], nbrb[u], semi[u])

    def drain_loads(u):
        dummy = src_hbm.at[pl.ds(0, CHUNK)]
        pltpu.make_async_copy(dummy, srcb[u], semi[u]).wait()
        pltpu.make_async_copy(dummy, nbrb[u], semi[u]).wait()

    def drain_scatters(u2, u4):
        pltpu.make_async_copy(rowsb[u2], acc.at[pl.ds(0, CHUNK)],
                              sems[u2]).wait()

        @pl.when(cid == u4 % 2)
        def _():
            pltpu.make_async_copy(ones_v, cnt.at[pl.ds(0, CHUNK)],
                                  sems[u2]).wait()

    fire_loads(0, 0)

    def outer(t, carry):
        for u in range(4):
            s = 4 * t + u

            @pl.when(jnp.logical_and(s >= 2, s - 2 <= nch - 1))
            def _(u2=(u - 2) % 2, u4=(u - 2) % 4):
                drain_scatters(u2, u4)

            @pl.when(s + 1 <= nch - 1)
            def _(u1=(u + 1) % 4, s=s):
                fire_loads(s + 1, u1)

            @pl.when(s <= nch - 1)
            def _(u=u, s=s):
                drain_loads(u)
                for j in range(CHUNK // 16):
                    sl = pl.ds(j * 16, 16)
                    idxb[u][sl] = nbrb[u][sl] * 2 + cid
                pltpu.async_copy(x2_hbm.at[idxb[u]], rowsb[u % 2], semg).wait()
                pltpu.async_copy(rowsb[u % 2], acc.at[srcb[u]], sems[u % 2],
                                 add=True)

                @pl.when(cid == u % 2)
                def _():
                    pltpu.async_copy(ones_v, cnt.at[srcb[u]],
                                     sems[u % 2], add=True)
        return carry

    lax.fori_loop(0, TMAX, outer, 0)
    plsc.subcore_barrier()

    my_rows = pl.ds(sid * RPT, RPT)
    pltpu.sync_copy(acc.at[my_rows], stage)
    pltpu.sync_copy(cnt.at[my_rows], st16)

    @pl.when(cid == 0)
    def _():
        pltpu.sync_copy(stage, sum_a_hbm.at[my_rows])
        pltpu.sync_copy(st16, cnt_a_hbm.at[my_rows])

    @pl.when(cid == 1)
    def _():
        pltpu.sync_copy(stage, sum_b_hbm.at[my_rows])
        pltpu.sync_copy(st16, cnt_b_hbm.at[my_rows])


@jax.jit
def _aggregate(x2, src, nbr, ones):
    mesh = plsc.VectorSubcoreMesh(core_axis_name="c", subcore_axis_name="s")
    idx_t = pltpu.VMEM((CHUNK,), jnp.int32)
    return pl.kernel(
        _agg_body,
        out_type=(
            jax.ShapeDtypeStruct((NP, DH), jnp.bfloat16),
            jax.ShapeDtypeStruct((NP, DH), jnp.bfloat16),
            jax.ShapeDtypeStruct((NP, 16), jnp.float32),
            jax.ShapeDtypeStruct((NP, 16), jnp.float32),
        ),
        mesh=mesh,
        compiler_params=pltpu.CompilerParams(use_tc_tiling_on_sc=False),
        scratch_types=[
            idx_t, idx_t, idx_t, idx_t,
            idx_t, idx_t, idx_t, idx_t,
            idx_t, idx_t, idx_t, idx_t,
            pltpu.VMEM((CHUNK, DH), jnp.bfloat16),
            pltpu.VMEM((CHUNK, DH), jnp.bfloat16),
            pltpu.VMEM((RPT, DH), jnp.bfloat16),
            pltpu.VMEM((CHUNK, 16), jnp.float32),
            pltpu.VMEM((RPT, 16), jnp.float32),
            pltpu.SemaphoreType.DMA, pltpu.SemaphoreType.DMA,
            pltpu.SemaphoreType.DMA, pltpu.SemaphoreType.DMA,
            pltpu.SemaphoreType.DMA,
            pltpu.SemaphoreType.DMA, pltpu.SemaphoreType.DMA,
            pltpu.VMEM_SHARED((NP, DH), jnp.bfloat16),
            pltpu.VMEM_SHARED((NP, 16), jnp.float32),
        ],
    )(x2, src, nbr, ones)


BLK = 1000


def _dense_body(x_ref, sa_ref, sb_ref, ca_ref, cb_ref,
                w1_ref, w2_ref, b1_ref, b2_ref, out_ref):
    x = x_ref[...]
    s = jnp.concatenate([sa_ref[...], sb_ref[...]], axis=1).astype(jnp.float32)
    cnt = ca_ref[:, 0:1] + cb_ref[:, 0:1]
    h = s / jnp.maximum(cnt, 1.0)
    dn = (((1,), (1,)), ((), ()))
    out_ref[...] = (
        lax.dot_general(x, w1_ref[...], dn, precision=lax.Precision.HIGHEST,
                        preferred_element_type=jnp.float32)
        + lax.dot_general(h, w2_ref[...], dn, precision=lax.Precision.HIGHEST,
                          preferred_element_type=jnp.float32)
        + b1_ref[...] + b2_ref[...]
    )


def _dense(x, sum_a, sum_b, cnt_a, cnt_b, W1, W2, b1, b2):
    return pl.pallas_call(
        _dense_body,
        grid=(N // BLK,),
        in_specs=[
            pl.BlockSpec((BLK, D), lambda i: (i, 0)),
            pl.BlockSpec((BLK, DH), lambda i: (i, 0)),
            pl.BlockSpec((BLK, DH), lambda i: (i, 0)),
            pl.BlockSpec((BLK, 16), lambda i: (i, 0)),
            pl.BlockSpec((BLK, 16), lambda i: (i, 0)),
            pl.BlockSpec((D, D), lambda i: (0, 0)),
            pl.BlockSpec((D, D), lambda i: (0, 0)),
            pl.BlockSpec((1, D), lambda i: (0, 0)),
            pl.BlockSpec((1, D), lambda i: (0, 0)),
        ],
        out_specs=pl.BlockSpec((BLK, D), lambda i: (i, 0)),
        out_shape=jax.ShapeDtypeStruct((N, D), jnp.float32),
    )(x, sum_a, sum_b, cnt_a, cnt_b, W1, W2, b1, b2)


@jax.jit
def _run(x, x2, src, nbr, ones, W1, b1, W2, b2):
    sum_a, sum_b, cnt_a, cnt_b = _aggregate(x2, src, nbr, ones)
    return _dense(x, sum_a, sum_b, cnt_a, cnt_b, W1, W2,
                  b1.reshape(1, D), b2.reshape(1, D))


def kernel(x, edge_index, W1, b1, W2, b2):
    src = edge_index[0]
    nbr = edge_index[1]
    x2 = x.astype(jnp.bfloat16).reshape(2 * N, DH)
    ones = jnp.ones((CHUNK, 16), jnp.float32)
    return _run(x, x2, src, nbr, ones, W1, b1, W2, b2)


# speedup vs baseline: 1.3878x; 1.3878x over previous
import functools

import jax
import jax.numpy as jnp
from jax import lax
from jax.experimental import pallas as pl
from jax.experimental.pallas import tpu as pltpu
from jax.experimental.pallas import tpu_sc as plsc

N = 10000
E = 320000
D = 128
DH = D // 2

NC = 2
NS = 16
EPT = E // NS
CHUNK = 80
K = 5
SUPE = K * CHUNK
NSUP = EPT // SUPE
NP = 10240
RPT = NP // NS
HPT = RPT // 2


def _agg_body(x2_hbm, src_hbm, nbr_hbm, ones_hbm,
              sum_a_hbm, sum_b_hbm, cnt_a_hbm, cnt_b_hbm,
              s0, s1, s2, s3, n0, n1, n2, n3, i0, i1, i2, i3,
              r0, r1, ones_v, st16,
              semi0, semi1, semi2, semi3, semg, sems0, sems1,
              acc, cnt):
    cid = lax.axis_index("c")
    sid = lax.axis_index("s")
    srcb = [s0, s1, s2, s3]
    nbrb = [n0, n1, n2, n3]
    idxb = [i0, i1, i2, i3]
    rowsb = [r0, r1]
    semi = [semi0, semi1, semi2, semi3]
    sems = [sems0, sems1]
    ebase = sid * EPT

    zv = jnp.zeros((16,), jnp.float32)
    zvb = jnp.zeros((32,), jnp.bfloat16)

    def zrow(i, carry):
        for j in range(DH // 32):
            r0[i, pl.ds(j * 32, 32)] = zvb
        return carry
    lax.fori_loop(0, HPT, zrow, 0)

    def zcnt(i, carry):
        st16[i] = zv
        return carry
    lax.fori_loop(0, RPT, zcnt, 0)
    pltpu.sync_copy(r0.at[pl.ds(0, HPT)], acc.at[pl.ds(sid * RPT, HPT)])
    pltpu.sync_copy(r0.at[pl.ds(0, HPT)], acc.at[pl.ds(sid * RPT + HPT, HPT)])
    pltpu.sync_copy(st16, cnt.at[pl.ds(sid * RPT, RPT)])
    pltpu.sync_copy(ones_hbm, ones_v)
    plsc.subcore_barrier()

    def fire_loads(s, u):
        base = ebase + s * SUPE
        for k in range(K):
            sl = pl.ds(base + k * CHUNK, CHUNK)
            pltpu.async_copy(src_hbm.at[sl], srcb[u].at[k], semi[u])
            pltpu.async_copy(nbr_hbm.at[sl], nbrb[u].at[k], semi[u])

    def drain_loads(u):
        dummy = src_hbm.at[pl.ds(0, CHUNK)]
        for k in range(K):
            pltpu.make_async_copy(dummy, srcb[u].at[k], semi[u]).wait()
            pltpu.make_async_copy(dummy, nbrb[u].at[k], semi[u]).wait()

    def drain_scatters(u2, u4):
        for k in range(K):
            pltpu.make_async_copy(rowsb[u2].at[pl.ds(0, CHUNK)],
                                  acc.at[pl.ds(0, CHUNK)], sems[u2]).wait()

            @pl.when(cid == (k + u4) % 2)
            def _():
                pltpu.make_async_copy(ones_v, cnt.at[pl.ds(0, CHUNK)],
                                      sems[u2]).wait()

    fire_loads(0, 0)

    def outer(t, carry):
        for u in range(4):
            s = 4 * t + u

            @pl.when(s >= 2)
            def _(u2=(u - 2) % 4, u4=(u - 2) % 4):
                drain_scatters(u2 % 2, u4)

            @pl.when(s + 1 <= NSUP - 1)
            def _(u1=(u + 1) % 4):
                fire_loads(s + 1, u1)

            @pl.when(s <= NSUP - 1)
            def _(u=u):
                drain_loads(u)
                for k in range(K):
                    for j in range(CHUNK // 16):
                        sl = pl.ds(j * 16, 16)
                        idxb[u][k, sl] = nbrb[u][k, sl] * 2 + cid
                descs = []
                for k in range(K):
                    descs.append(pltpu.async_copy(
                        x2_hbm.at[idxb[u].at[k]],
                        rowsb[u % 2].at[pl.ds(k * CHUNK, CHUNK)], semg))
                for k in range(K):
                    descs[k].wait()
                    pltpu.async_copy(rowsb[u % 2].at[pl.ds(k * CHUNK, CHUNK)],
                                     acc.at[srcb[u].at[k]], sems[u % 2],
                                     add=True)

                    @pl.when(cid == (k + u) % 2)
                    def _(k=k):
                        pltpu.async_copy(ones_v, cnt.at[srcb[u].at[k]],
                                         sems[u % 2], add=True)
        return carry

    lax.fori_loop(0, (NSUP + 2 + 3) // 4, outer, 0)
    plsc.subcore_barrier()

    for piece in range(2):
        rows = pl.ds(sid * RPT + piece * HPT, HPT)
        pltpu.sync_copy(acc.at[rows], rowsb[piece].at[pl.ds(0, HPT)])
    pltpu.sync_copy(cnt.at[pl.ds(sid * RPT, RPT)], st16)

    @pl.when(cid == 0)
    def _():
        for piece in range(2):
            rows = pl.ds(sid * RPT + piece * HPT, HPT)
            pltpu.sync_copy(rowsb[piece].at[pl.ds(0, HPT)], sum_a_hbm.at[rows])
        pltpu.sync_copy(st16, cnt_a_hbm.at[pl.ds(sid * RPT, RPT)])

    @pl.when(cid == 1)
    def _():
        for piece in range(2):
            rows = pl.ds(sid * RPT + piece * HPT, HPT)
            pltpu.sync_copy(rowsb[piece].at[pl.ds(0, HPT)], sum_b_hbm.at[rows])
        pltpu.sync_copy(st16, cnt_b_hbm.at[pl.ds(sid * RPT, RPT)])


@jax.jit
def _aggregate(x2, src, nbr, ones):
    mesh = plsc.VectorSubcoreMesh(core_axis_name="c", subcore_axis_name="s")
    idx_t = pltpu.VMEM((K, CHUNK), jnp.int32)
    return pl.kernel(
        _agg_body,
        out_type=(
            jax.ShapeDtypeStruct((NP, DH), jnp.bfloat16),
            jax.ShapeDtypeStruct((NP, DH), jnp.bfloat16),
            jax.ShapeDtypeStruct((NP, 16), jnp.float32),
            jax.ShapeDtypeStruct((NP, 16), jnp.float32),
        ),
        mesh=mesh,
        compiler_params=pltpu.CompilerParams(use_tc_tiling_on_sc=False),
        scratch_types=[
            idx_t, idx_t, idx_t, idx_t,
            idx_t, idx_t, idx_t, idx_t,
            idx_t, idx_t, idx_t, idx_t,
            pltpu.VMEM((SUPE, DH), jnp.bfloat16),
            pltpu.VMEM((SUPE, DH), jnp.bfloat16),
            pltpu.VMEM((CHUNK, 16), jnp.float32),
            pltpu.VMEM((RPT, 16), jnp.float32),
            pltpu.SemaphoreType.DMA, pltpu.SemaphoreType.DMA,
            pltpu.SemaphoreType.DMA, pltpu.SemaphoreType.DMA,
            pltpu.SemaphoreType.DMA,
            pltpu.SemaphoreType.DMA, pltpu.SemaphoreType.DMA,
            pltpu.VMEM_SHARED((NP, DH), jnp.bfloat16),
            pltpu.VMEM_SHARED((NP, 16), jnp.float32),
        ],
    )(x2, src, nbr, ones)


BLK = 1000


def _dense_body(x_ref, sa_ref, sb_ref, ca_ref, cb_ref,
                w1_ref, w2_ref, b1_ref, b2_ref, out_ref):
    x = x_ref[...]
    s = jnp.concatenate([sa_ref[...], sb_ref[...]], axis=1).astype(jnp.float32)
    cnt = ca_ref[:, 0:1] + cb_ref[:, 0:1]
    h = s / jnp.maximum(cnt, 1.0)
    dn = (((1,), (1,)), ((), ()))
    out_ref[...] = (
        lax.dot_general(x, w1_ref[...], dn, precision=lax.Precision.HIGHEST,
                        preferred_element_type=jnp.float32)
        + lax.dot_general(h, w2_ref[...], dn, precision=lax.Precision.HIGHEST,
                          preferred_element_type=jnp.float32)
        + b1_ref[...] + b2_ref[...]
    )


def _dense(x, sum_a, sum_b, cnt_a, cnt_b, W1, W2, b1, b2):
    return pl.pallas_call(
        _dense_body,
        grid=(N // BLK,),
        in_specs=[
            pl.BlockSpec((BLK, D), lambda i: (i, 0)),
            pl.BlockSpec((BLK, DH), lambda i: (i, 0)),
            pl.BlockSpec((BLK, DH), lambda i: (i, 0)),
            pl.BlockSpec((BLK, 16), lambda i: (i, 0)),
            pl.BlockSpec((BLK, 16), lambda i: (i, 0)),
            pl.BlockSpec((D, D), lambda i: (0, 0)),
            pl.BlockSpec((D, D), lambda i: (0, 0)),
            pl.BlockSpec((1, D), lambda i: (0, 0)),
            pl.BlockSpec((1, D), lambda i: (0, 0)),
        ],
        out_specs=pl.BlockSpec((BLK, D), lambda i: (i, 0)),
        out_shape=jax.ShapeDtypeStruct((N, D), jnp.float32),
    )(x, sum_a, sum_b, cnt_a, cnt_b, W1, W2, b1, b2)


@jax.jit
def _run(x, x2, src, nbr, ones, W1, b1, W2, b2):
    sum_a, sum_b, cnt_a, cnt_b = _aggregate(x2, src, nbr, ones)
    return _dense(x, sum_a, sum_b, cnt_a, cnt_b, W1, W2,
                  b1.reshape(1, D), b2.reshape(1, D))


def kernel(x, edge_index, W1, b1, W2, b2):
    src = edge_index[0]
    nbr = edge_index[1]
    x2 = x.astype(jnp.bfloat16).reshape(2 * N, DH)
    ones = jnp.ones((CHUNK, 16), jnp.float32)
    return _run(x, x2, src, nbr, ones, W1, b1, W2, b2)


# speedup vs baseline: 1.4516x; 1.0459x over previous
import functools

import jax
import jax.numpy as jnp
from jax import lax
from jax.experimental import pallas as pl
from jax.experimental.pallas import tpu as pltpu
from jax.experimental.pallas import tpu_sc as plsc

N = 10000
E = 320000
D = 128
DH = D // 2

NC = 2
NS = 16
EPT = E // NS
CHUNK = 80
K = 5
SUPE = K * CHUNK
NSUP = EPT // SUPE
NP = 10240
RPT = NP // NS
HPT = RPT // 2


def _agg_body(x2_hbm, src_hbm, nbr_hbm, ones_hbm,
              sum_a_hbm, sum_b_hbm, cnt_a_hbm, cnt_b_hbm,
              s0, s1, s2, s3, n0, n1, n2, n3, i0, i1, i2, i3,
              r0, r1, ones_v, st16,
              semi0, semi1, semi2, semi3, semg, sems0, sems1,
              acc, cnt):
    cid = lax.axis_index("c")
    sid = lax.axis_index("s")
    srcb = [s0, s1, s2, s3]
    nbrb = [n0, n1, n2, n3]
    idxb = [i0, i1, i2, i3]
    rowsb = [r0, r1]
    semi = [semi0, semi1, semi2, semi3]
    sems = [sems0, sems1]
    ebase = sid * EPT

    zv = jnp.zeros((16,), jnp.float32)
    zvb = jnp.zeros((32,), jnp.bfloat16)

    def zrow(i, carry):
        for j in range(DH // 32):
            r0[i, pl.ds(j * 32, 32)] = zvb
        return carry
    lax.fori_loop(0, HPT, zrow, 0)

    def zcnt(i, carry):
        st16[i] = zv
        return carry
    lax.fori_loop(0, RPT, zcnt, 0)
    pltpu.sync_copy(r0.at[pl.ds(0, HPT)], acc.at[pl.ds(sid * RPT, HPT)])
    pltpu.sync_copy(r0.at[pl.ds(0, HPT)], acc.at[pl.ds(sid * RPT + HPT, HPT)])
    pltpu.sync_copy(st16, cnt.at[pl.ds(sid * RPT, RPT)])
    pltpu.sync_copy(ones_hbm, ones_v)
    plsc.subcore_barrier()

    def fire_loads(s, u):
        base = ebase + s * SUPE
        for k in range(K):
            sl = pl.ds(base + k * CHUNK, CHUNK)
            pltpu.async_copy(src_hbm.at[sl], srcb[u].at[k], semi[u])
            pltpu.async_copy(nbr_hbm.at[sl], nbrb[u].at[k], semi[u])

    def drain_loads(u):
        dummy = src_hbm.at[pl.ds(0, CHUNK)]
        for k in range(K):
            pltpu.make_async_copy(dummy, srcb[u].at[k], semi[u]).wait()
            pltpu.make_async_copy(dummy, nbrb[u].at[k], semi[u]).wait()

    def drain_scatters(u2, u4):
        for k in range(K):
            pltpu.make_async_copy(rowsb[u2].at[pl.ds(0, CHUNK)],
                                  acc.at[pl.ds(0, CHUNK)], sems[u2]).wait()

            @pl.when(cid == (k + u4) % 2)
            def _():
                pltpu.make_async_copy(ones_v, cnt.at[pl.ds(0, CHUNK)],
                                      sems[u2]).wait()

    fire_loads(0, 0)

    def outer(t, carry):
        for u in range(4):
            s = 4 * t + u

            @pl.when(s >= 2)
            def _(u2=(u - 2) % 4, u4=(u - 2) % 4):
                drain_scatters(u2 % 2, u4)

            @pl.when(s + 1 <= NSUP - 1)
            def _(u1=(u + 1) % 4):
                fire_loads(s + 1, u1)

            @pl.when(s <= NSUP - 1)
            def _(u=u):
                drain_loads(u)
                for k in range(K):
                    for j in range(CHUNK // 16):
                        sl = pl.ds(j * 16, 16)
                        idxb[u][k, sl] = nbrb[u][k, sl] * 2 + cid
                descs = []
                for k in range(K):
                    descs.append(pltpu.async_copy(
                        x2_hbm.at[idxb[u].at[k]],
                        rowsb[u % 2].at[pl.ds(k * CHUNK, CHUNK)], semg))
                for k in range(K):
                    descs[k].wait()
                    pltpu.async_copy(rowsb[u % 2].at[pl.ds(k * CHUNK, CHUNK)],
                                     acc.at[srcb[u].at[k]], sems[u % 2],
                                     add=True)

                    @pl.when(cid == (k + u) % 2)
                    def _(k=k):
                        pltpu.async_copy(ones_v, cnt.at[srcb[u].at[k]],
                                         sems[u % 2], add=True)
        return carry

    lax.fori_loop(0, (NSUP + 2 + 3) // 4, outer, 0)
    plsc.subcore_barrier()

    for piece in range(2):
        rows = pl.ds(sid * RPT + piece * HPT, HPT)
        pltpu.sync_copy(acc.at[rows], rowsb[piece].at[pl.ds(0, HPT)])
    pltpu.sync_copy(cnt.at[pl.ds(sid * RPT, RPT)], st16)

    @pl.when(cid == 0)
    def _():
        for piece in range(2):
            rows = pl.ds(sid * RPT + piece * HPT, HPT)
            pltpu.sync_copy(rowsb[piece].at[pl.ds(0, HPT)], sum_a_hbm.at[rows])
        pltpu.sync_copy(st16, cnt_a_hbm.at[pl.ds(sid * RPT, RPT)])

    @pl.when(cid == 1)
    def _():
        for piece in range(2):
            rows = pl.ds(sid * RPT + piece * HPT, HPT)
            pltpu.sync_copy(rowsb[piece].at[pl.ds(0, HPT)], sum_b_hbm.at[rows])
        pltpu.sync_copy(st16, cnt_b_hbm.at[pl.ds(sid * RPT, RPT)])


@jax.jit
def _aggregate(x2, src, nbr, ones):
    mesh = plsc.VectorSubcoreMesh(core_axis_name="c", subcore_axis_name="s")
    idx_t = pltpu.VMEM((K, CHUNK), jnp.int32)
    return pl.kernel(
        _agg_body,
        out_type=(
            jax.ShapeDtypeStruct((NP, DH), jnp.bfloat16),
            jax.ShapeDtypeStruct((NP, DH), jnp.bfloat16),
            jax.ShapeDtypeStruct((NP, 16), jnp.float32),
            jax.ShapeDtypeStruct((NP, 16), jnp.float32),
        ),
        mesh=mesh,
        compiler_params=pltpu.CompilerParams(use_tc_tiling_on_sc=False),
        scratch_types=[
            idx_t, idx_t, idx_t, idx_t,
            idx_t, idx_t, idx_t, idx_t,
            idx_t, idx_t, idx_t, idx_t,
            pltpu.VMEM((SUPE, DH), jnp.bfloat16),
            pltpu.VMEM((SUPE, DH), jnp.bfloat16),
            pltpu.VMEM((CHUNK, 16), jnp.float32),
            pltpu.VMEM((RPT, 16), jnp.float32),
            pltpu.SemaphoreType.DMA, pltpu.SemaphoreType.DMA,
            pltpu.SemaphoreType.DMA, pltpu.SemaphoreType.DMA,
            pltpu.SemaphoreType.DMA,
            pltpu.SemaphoreType.DMA, pltpu.SemaphoreType.DMA,
            pltpu.VMEM_SHARED((NP, DH), jnp.bfloat16),
            pltpu.VMEM_SHARED((NP, 16), jnp.float32),
        ],
    )(x2, src, nbr, ones)


BLK = 2000


def _dense_body(x_ref, sa_ref, sb_ref, ca_ref, cb_ref,
                w1_ref, w2_ref, b1_ref, b2_ref, out_ref):
    x = x_ref[...]
    s = jnp.concatenate([sa_ref[...], sb_ref[...]], axis=1).astype(jnp.float32)
    cnt = ca_ref[:, 0:1] + cb_ref[:, 0:1]
    h = s / jnp.maximum(cnt, 1.0)
    dn = (((1,), (1,)), ((), ()))
    out_ref[...] = (
        lax.dot_general(x, w1_ref[...], dn, precision=lax.Precision.HIGHEST,
                        preferred_element_type=jnp.float32)
        + lax.dot_general(h, w2_ref[...], dn, precision=lax.Precision.HIGHEST,
                          preferred_element_type=jnp.float32)
        + b1_ref[...] + b2_ref[...]
    )


def _dense(x, sum_a, sum_b, cnt_a, cnt_b, W1, W2, b1, b2):
    return pl.pallas_call(
        _dense_body,
        grid=(N // BLK,),
        in_specs=[
            pl.BlockSpec((BLK, D), lambda i: (i, 0)),
            pl.BlockSpec((BLK, DH), lambda i: (i, 0)),
            pl.BlockSpec((BLK, DH), lambda i: (i, 0)),
            pl.BlockSpec((BLK, 16), lambda i: (i, 0)),
            pl.BlockSpec((BLK, 16), lambda i: (i, 0)),
            pl.BlockSpec((D, D), lambda i: (0, 0)),
            pl.BlockSpec((D, D), lambda i: (0, 0)),
            pl.BlockSpec((1, D), lambda i: (0, 0)),
            pl.BlockSpec((1, D), lambda i: (0, 0)),
        ],
        out_specs=pl.BlockSpec((BLK, D), lambda i: (i, 0)),
        out_shape=jax.ShapeDtypeStruct((N, D), jnp.float32),
    )(x, sum_a, sum_b, cnt_a, cnt_b, W1, W2, b1, b2)


@jax.jit
def _run(x, x2, src, nbr, ones, W1, b1, W2, b2):
    sum_a, sum_b, cnt_a, cnt_b = _aggregate(x2, src, nbr, ones)
    return _dense(x, sum_a, sum_b, cnt_a, cnt_b, W1, W2,
                  b1.reshape(1, D), b2.reshape(1, D))


def kernel(x, edge_index, W1, b1, W2, b2):
    src = edge_index[0]
    nbr = edge_index[1]
    x2 = x.astype(jnp.bfloat16).reshape(2 * N, DH)
    ones = jnp.ones((CHUNK, 16), jnp.float32)
    return _run(x, x2, src, nbr, ones, W1, b1, W2, b2)
